# Initial kernel scaffold; baseline (speedup 1.0000x reference)
import jax
import jax.numpy as jnp
from jax.experimental import pallas as pl


def kernel(H, u_s, gather_idx, gather_idx2, u_s_gather_idx):
    raise NotImplementedError("write your pallas kernel here")


# speedup vs baseline: 103.7473x; 103.7473x over previous
import jax
import jax.numpy as jnp
from jax import lax
from jax.experimental import pallas as pl
from jax.experimental.pallas import tpu as pltpu
from jax.experimental.pallas import tpu_sc as plsc

B = 4
D = 900000
S = 100000
NP = 200000
N_ENT = NP * 9

NCORES = 2
NTILES = 16
PT = 12544
NP_PAD = PT * NTILES
EPT = PT * 9
E_PAD = NP_PAD * 9
G_ENT = 2304
N_GROUPS = EPT // G_ENT
UCH = 1568

HALF = 1800000
TRASH_BASE = 1800000
TRASH_MASK = 2047
CW = 3600
C_WORDS = 1803600
NZCH = C_WORDS // CW
NWCH = HALF // CW

_GDN = lax.GatherDimensionNumbers(
    offset_dims=(), collapsed_slice_dims=(0,), start_index_map=(0,))


def _vperm(x, idx):
    return lax.gather(x, idx[:, None], _GDN, (1,),
                      mode=lax.GatherScatterMode.PROMISE_IN_BOUNDS)


def _sc_body(lin1, lin2, ulin, us, h, out, fnhb,
             inbuf, idxb, valb, ulbc, uvbc, fnegc, cbuf, hbuf, C, sem):
    sc = lax.axis_index("c")
    s = lax.axis_index("s")
    base = sc * HALF

    lane = lax.iota(jnp.int32, 16)
    zero16 = lane.astype(jnp.float32) * 0.0

    def _z1(i, c):
        cbuf[pl.ds(i * 16, 16)] = zero16
        return c
    lax.fori_loop(0, CW // 16, _z1, 0)

    def _z2(i, c):
        k = i * 32 + s * 2

        @pl.when(k < NZCH)
        def _():
            pltpu.sync_copy(cbuf, C.at[pl.ds(k * CW, CW)])

        @pl.when(k + 1 < NZCH)
        def _():
            pltpu.sync_copy(cbuf, C.at[pl.ds((k + 1) * CW, CW)])
        return c
    lax.fori_loop(0, 16, _z2, 0)

    for i in range(NTILES // 2):
        ub = s * PT + i * UCH
        pltpu.sync_copy(ulin.at[pl.ds(ub, UCH)], ulbc)
        pltpu.async_copy(us.at[ulbc], uvbc, sem).wait()

        def _fk(r, c):
            u = uvbc[pl.ds(r * 16, 16)]
            uvbc[pl.ds(r * 16, 16)] = jnp.where(
                u < 0.0, jnp.float32(-2.0), jnp.float32(0.0))
            return c
        lax.fori_loop(0, UCH // 16, _fk, 0)
        pltpu.sync_copy(uvbc, fnhb.at[pl.ds(sc * NP_PAD + ub, UCH)])

    plsc.subcore_barrier()

    permj, flipj = [], []
    for j in range(9):
        t = lane + 16 * j
        p = lax.shift_right_logical(t * 57, 9)
        e = t - p * 9
        permj.append(p)
        flipj.append(jnp.where((e == 2) | (e == 5) | (e == 6) | (e == 7),
                               jnp.float32(1.0), jnp.float32(0.0)))

    def _scatter_array(lin_hbm):
        def _grp(g, c):
            pltpu.sync_copy(lin_hbm.at[pl.ds(s * EPT + g * G_ENT, G_ENT)],
                            inbuf)
            pltpu.sync_copy(fnhb.at[pl.ds(sc * NP_PAD + s * PT + g * 256, 256)], fnegc)
            def _chunk(cc, c2):
                f16 = fnegc[pl.ds(cc * 16, 16)]
                tb = cc * 144
                for j in range(9):
                    t = tb + j * 16
                    lv = inbuf[pl.ds(t, 16)]
                    off = lv - base
                    inr = (off >= 0) & (off < HALF)
                    tr = TRASH_BASE + lax.bitwise_and(lv, TRASH_MASK)
                    idxb[pl.ds(t, 16)] = jnp.where(inr, off, tr)
                    v = _vperm(f16, permj[j]) * flipj[j] + 1.0
                    valb[pl.ds(t, 16)] = v
                return c2
            lax.fori_loop(0, 16, _chunk, 0)
            pltpu.sync_copy(valb, C.at[idxb], add=True)
            return c
        lax.fori_loop(0, N_GROUPS, _grp, 0)

    _scatter_array(lin1)
    _scatter_array(lin2)

    plsc.subcore_barrier()

    def _wo(i, c):
        k = i * 32 + s * 2

        def _one(kk):
            st = kk * CW
            pltpu.sync_copy(C.at[pl.ds(st, CW)], cbuf)
            pltpu.sync_copy(h.at[pl.ds(base + st, CW)], hbuf)

            def _m(q, c2):
                q16 = q * 16
                cbuf[pl.ds(q16, 16)] = (cbuf[pl.ds(q16, 16)]
                                        * hbuf[pl.ds(q16, 16)])
                return c2
            lax.fori_loop(0, CW // 16, _m, 0)
            pltpu.sync_copy(cbuf, out.at[pl.ds(base + st, CW)])

        @pl.when(k < NWCH)
        def _():
            _one(k)

        @pl.when(k + 1 < NWCH)
        def _():
            _one(k + 1)
        return c
    lax.fori_loop(0, 16, _wo, 0)


@jax.jit
def _sc_call(lin1, lin2, ulin, us_flat, h_flat):
    return pl.kernel(
        _sc_body,
        out_type=(jax.ShapeDtypeStruct((B * D,), jnp.float32),
                  jax.ShapeDtypeStruct((NCORES * NP_PAD,), jnp.float32)),
        mesh=plsc.VectorSubcoreMesh(core_axis_name="c", subcore_axis_name="s",
                                    num_cores=NCORES, num_subcores=NTILES),
        scratch_types=[
            pltpu.VMEM((G_ENT,), jnp.int32),
            pltpu.VMEM((G_ENT,), jnp.int32),
            pltpu.VMEM((G_ENT,), jnp.float32),
            pltpu.VMEM((UCH,), jnp.int32),
            pltpu.VMEM((UCH,), jnp.float32),
            pltpu.VMEM((256,), jnp.float32),
            pltpu.VMEM((CW,), jnp.float32),
            pltpu.VMEM((CW,), jnp.float32),
            pltpu.VMEM_SHARED((C_WORDS,), jnp.float32),
            pltpu.SemaphoreType.DMA,
        ],
    )(lin1, lin2, ulin, us_flat, h_flat)


def kernel(H, u_s, gather_idx, gather_idx2, u_s_gather_idx):
    lin1 = gather_idx[:, 0] * D + gather_idx[:, 1]
    lin2 = gather_idx2[:, 0] * D + gather_idx2[:, 1]
    npad = E_PAD - N_ENT
    sent = (jnp.int32(4194304)
            + (jnp.arange(npad, dtype=jnp.int32) & TRASH_MASK))
    lin1 = jnp.concatenate([lin1.astype(jnp.int32), sent])
    lin2 = jnp.concatenate([lin2.astype(jnp.int32), sent])
    ul = u_s_gather_idx[:, 0] * S + u_s_gather_idx[:, 1]
    ul = jnp.concatenate(
        [ul.astype(jnp.int32), jnp.zeros((NP_PAD - NP,), jnp.int32)])
    out, _ = _sc_call(lin1, lin2, ul, u_s.reshape(-1), H.reshape(-1))
    return out.reshape(B, D)


# speedup vs baseline: 151.8231x; 1.4634x over previous
import jax
import jax.numpy as jnp
from jax import lax
from jax.experimental import pallas as pl
from jax.experimental.pallas import tpu as pltpu
from jax.experimental.pallas import tpu_sc as plsc

B = 4
D = 900000
S = 100000
NP = 200000
N_ENT = NP * 9

NCORES = 2
NTILES = 16
PT = 12544
NP_PAD = PT * NTILES
EPT = PT * 9
E_PAD = NP_PAD * 9
G_ENT = 1152
FL = G_ENT // 9
N_GROUPS = EPT // G_ENT
UCH = 1568

HALF = 1800000
TRASH_BASE = 1800000
TRASH_MASK = 2047
CW = 1600
C_WORDS = 1803200
NZCH = C_WORDS // CW
NWCH = HALF // CW

_GDN = lax.GatherDimensionNumbers(
    offset_dims=(), collapsed_slice_dims=(0,), start_index_map=(0,))


def _vperm(x, idx):
    return lax.gather(x, idx[:, None], _GDN, (1,),
                      mode=lax.GatherScatterMode.PROMISE_IN_BOUNDS)


def _sc_body(lin1, lin2, ulin, us, h, out, fnhb,
             ib0, ib1, xb0, xb1, vb0, vb1, fc0, fc1, ulbc, uvbc,
             cbA, cbB, hbA, hbB, C,
             semg, semI0, semI1, semF0, semF1, semS0, semS1,
             semC0, semC1, semH0, semH1, semO0, semO1):
    sc = lax.axis_index("c")
    s = lax.axis_index("s")
    base = sc * HALF

    lane = lax.iota(jnp.int32, 16)
    zero16 = lane.astype(jnp.float32) * 0.0

    def _z1(i, c):
        cbA[pl.ds(i * 16, 16)] = zero16
        return c
    lax.fori_loop(0, CW // 16, _z1, 0)

    def _z2(i, c):
        k = i * 32 + s * 2

        @pl.when(k < NZCH)
        def _():
            pltpu.sync_copy(cbA, C.at[pl.ds(k * CW, CW)])

        @pl.when(k + 1 < NZCH)
        def _():
            pltpu.sync_copy(cbA, C.at[pl.ds((k + 1) * CW, CW)])
        return c
    lax.fori_loop(0, (NZCH + 31) // 32, _z2, 0)

    fbase = sc * NP_PAD + s * PT
    for i in range(PT // UCH):
        ub = s * PT + i * UCH
        pltpu.sync_copy(ulin.at[pl.ds(ub, UCH)], ulbc)
        pltpu.async_copy(us.at[ulbc], uvbc, semg).wait()

        def _fk(r, c):
            u = uvbc[pl.ds(r * 16, 16)]
            uvbc[pl.ds(r * 16, 16)] = jnp.where(
                u < 0.0, jnp.float32(-2.0), jnp.float32(0.0))
            return c
        lax.fori_loop(0, UCH // 16, _fk, 0)
        pltpu.sync_copy(uvbc, fnhb.at[pl.ds(sc * NP_PAD + ub, UCH)])

    plsc.subcore_barrier()

    permj, flipj = [], []
    for j in range(9):
        t = lane + 16 * j
        p = lax.shift_right_logical(t * 57, 9)
        e = t - p * 9
        permj.append(p)
        flipj.append(jnp.where((e == 2) | (e == 5) | (e == 6) | (e == 7),
                               jnp.float32(1.0), jnp.float32(0.0)))

    def _scatter_array(lin_hbm):
        ebase = s * EPT

        def _issue_in(g, ib, fc, semI, semF):
            pltpu.async_copy(lin_hbm.at[pl.ds(ebase + g * G_ENT, G_ENT)],
                             ib, semI)
            pltpu.async_copy(fnhb.at[pl.ds(fbase + g * FL, FL)], fc, semF)

        def _do_group(g, ib, xb, vb, fc, semI, semF, semS):
            pltpu.make_async_copy(lin_hbm.at[pl.ds(0, G_ENT)], ib,
                                  semI).wait()
            pltpu.make_async_copy(fnhb.at[pl.ds(0, FL)], fc, semF).wait()

            @pl.when(g >= 2)
            def _():
                pltpu.make_async_copy(vb, C.at[xb], semS).wait()

            def _chunk(cc, c2):
                f16 = fc[pl.ds(cc * 16, 16)]
                tb = cc * 144
                for j in range(9):
                    t = tb + j * 16
                    lv = ib[pl.ds(t, 16)]
                    off = lv - base
                    inr = (off >= 0) & (off < HALF)
                    tr = TRASH_BASE + lax.bitwise_and(lv, TRASH_MASK)
                    xb[pl.ds(t, 16)] = jnp.where(inr, off, tr)
                    v = _vperm(f16, permj[j]) * flipj[j] + 1.0
                    vb[pl.ds(t, 16)] = v
                return c2
            lax.fori_loop(0, G_ENT // 144, _chunk, 0)
            pltpu.async_copy(vb, C.at[xb], semS, add=True)

            @pl.when(g + 2 < N_GROUPS)
            def _():
                _issue_in(g + 2, ib, fc, semI, semF)

        _issue_in(0, ib0, fc0, semI0, semF0)
        _issue_in(1, ib1, fc1, semI1, semF1)

        def _gg(gg, c):
            g = gg * 2
            _do_group(g, ib0, xb0, vb0, fc0, semI0, semF0, semS0)
            _do_group(g + 1, ib1, xb1, vb1, fc1, semI1, semF1, semS1)
            return c
        lax.fori_loop(0, N_GROUPS // 2, _gg, 0)

        pltpu.make_async_copy(vb0, C.at[xb0], semS0).wait()
        pltpu.make_async_copy(vb1, C.at[xb1], semS1).wait()

    _scatter_array(lin1)
    _scatter_array(lin2)

    plsc.subcore_barrier()

    def _mul(cb, hb):
        def _m(q, c2):
            q16 = q * 16
            cb[pl.ds(q16, 16)] = cb[pl.ds(q16, 16)] * hb[pl.ds(q16, 16)]
            return c2
        lax.fori_loop(0, CW // 16, _m, 0)

    def _wo(i, c):
        k0 = i * 32 + s * 2
        k1 = k0 + 1

        @pl.when(k0 < NWCH)
        def _():
            @pl.when(i > 0)
            def _():
                pltpu.make_async_copy(cbA, out.at[pl.ds(0, CW)], semO0).wait()
            pltpu.async_copy(C.at[pl.ds(k0 * CW, CW)], cbA, semC0)
            pltpu.async_copy(h.at[pl.ds(base + k0 * CW, CW)], hbA, semH0)

        @pl.when(k1 < NWCH)
        def _():
            @pl.when(i > 0)
            def _():
                pltpu.make_async_copy(cbB, out.at[pl.ds(0, CW)], semO1).wait()
            pltpu.async_copy(C.at[pl.ds(k1 * CW, CW)], cbB, semC1)
            pltpu.async_copy(h.at[pl.ds(base + k1 * CW, CW)], hbB, semH1)

        @pl.when(k0 < NWCH)
        def _():
            pltpu.make_async_copy(C.at[pl.ds(0, CW)], cbA, semC0).wait()
            pltpu.make_async_copy(h.at[pl.ds(0, CW)], hbA, semH0).wait()
            _mul(cbA, hbA)
            pltpu.async_copy(cbA, out.at[pl.ds(base + k0 * CW, CW)], semO0)

        @pl.when(k1 < NWCH)
        def _():
            pltpu.make_async_copy(C.at[pl.ds(0, CW)], cbB, semC1).wait()
            pltpu.make_async_copy(h.at[pl.ds(0, CW)], hbB, semH1).wait()
            _mul(cbB, hbB)
            pltpu.async_copy(cbB, out.at[pl.ds(base + k1 * CW, CW)], semO1)
        return c
    lax.fori_loop(0, (NWCH + 31) // 32, _wo, 0)

    pltpu.make_async_copy(cbA, out.at[pl.ds(0, CW)], semO0).wait()
    pltpu.make_async_copy(cbB, out.at[pl.ds(0, CW)], semO1).wait()


@jax.jit
def _sc_call(lin1, lin2, ulin, us_flat, h_flat):
    dma = pltpu.SemaphoreType.DMA
    return pl.kernel(
        _sc_body,
        out_type=(jax.ShapeDtypeStruct((B * D,), jnp.float32),
                  jax.ShapeDtypeStruct((NCORES * NP_PAD,), jnp.float32)),
        mesh=plsc.VectorSubcoreMesh(core_axis_name="c", subcore_axis_name="s",
                                    num_cores=NCORES, num_subcores=NTILES),
        scratch_types=[
            pltpu.VMEM((G_ENT,), jnp.int32),
            pltpu.VMEM((G_ENT,), jnp.int32),
            pltpu.VMEM((G_ENT,), jnp.int32),
            pltpu.VMEM((G_ENT,), jnp.int32),
            pltpu.VMEM((G_ENT,), jnp.float32),
            pltpu.VMEM((G_ENT,), jnp.float32),
            pltpu.VMEM((FL,), jnp.float32),
            pltpu.VMEM((FL,), jnp.float32),
            pltpu.VMEM((UCH,), jnp.int32),
            pltpu.VMEM((UCH,), jnp.float32),
            pltpu.VMEM((CW,), jnp.float32),
            pltpu.VMEM((CW,), jnp.float32),
            pltpu.VMEM((CW,), jnp.float32),
            pltpu.VMEM((CW,), jnp.float32),
            pltpu.VMEM_SHARED((C_WORDS,), jnp.float32),
            dma, dma, dma, dma, dma, dma, dma,
            dma, dma, dma, dma, dma, dma,
        ],
    )(lin1, lin2, ulin, us_flat, h_flat)


def kernel(H, u_s, gather_idx, gather_idx2, u_s_gather_idx):
    lin1 = gather_idx[:, 0] * D + gather_idx[:, 1]
    lin2 = gather_idx2[:, 0] * D + gather_idx2[:, 1]
    npad = E_PAD - N_ENT
    sent = (jnp.int32(4194304)
            + (jnp.arange(npad, dtype=jnp.int32) & TRASH_MASK))
    lin1 = jnp.concatenate([lin1.astype(jnp.int32), sent])
    lin2 = jnp.concatenate([lin2.astype(jnp.int32), sent])
    ul = u_s_gather_idx[:, 0] * S + u_s_gather_idx[:, 1]
    ul = jnp.concatenate(
        [ul.astype(jnp.int32), jnp.zeros((NP_PAD - NP,), jnp.int32)])
    out, _ = _sc_call(lin1, lin2, ul, u_s.reshape(-1), H.reshape(-1))
    return out.reshape(B, D)
